# baseline (speedup 1.0000x reference)
import functools

import jax
import jax.numpy as jnp
import numpy as np
from jax import lax
from jax.experimental import pallas as pl
from jax.experimental.pallas import tpu as pltpu
from jax.experimental.pallas import tpu_sc as plsc

N_ATOMS = 2048
BOX = 24.0
R_CUT = 5.0
N_GAUSS = 50
HID = 256
NFILT = 256
SHIFT = float(np.log(2.0))

EA_PAD = 64
EDGE_BLK = 2048

NC = 2
NS = 16
NW = NC * NS
APW = N_ATOMS // NW
WCAP = 6144
NEDIR = NW * WCAP
CH = 128
ACC_ROWS = APW + 8
TRASH_XH = N_ATOMS
XH_ROWS = N_ATOMS + 128

R2CUT = R_CUT * R_CUT


def _ssp(x):
    return jax.nn.softplus(x) - SHIFT


def _nbr_body(d2_hbm, jidx_hbm, d2e_hbm, dst_hbm, cnt_hbm,
              d2row_v, jbuf_v, d2buf_v, dstbuf_v, cnt_v):
    cid = lax.axis_index("c")
    sid = lax.axis_index("s")
    wid = sid * NC + cid
    lanes = lax.iota(jnp.int32, 16)

    def pre(k, _):
        sl = pl.ds(k * 16, 16)
        jbuf_v[sl] = jnp.full((16,), TRASH_XH, jnp.int32)
        d2buf_v[sl] = jnp.zeros((16,), jnp.float32)
        dstbuf_v[sl] = jnp.full((16,), APW * NFILT, jnp.int32)
        return 0
    lax.fori_loop(0, WCAP // 16, pre, 0)

    def per_row(a, ptr_vec):
        i = wid * APW + a
        pltpu.sync_copy(d2_hbm.at[i], d2row_v)

        def per_vec(v, ptr_vec):
            d2v = d2row_v[pl.ds(v * 16, 16)]
            jv = lanes + v * 16
            m = (d2v < R2CUT) & (jv != i)
            pos = ptr_vec + plsc.cumsum(m.astype(jnp.int32)) - 1
            plsc.store_scatter(jbuf_v, [pos], jv, mask=m)
            plsc.store_scatter(d2buf_v, [pos], d2v, mask=m)
            plsc.store_scatter(dstbuf_v, [pos],
                               jnp.full((16,), a * NFILT, jnp.int32),
                               mask=m)
            return ptr_vec + plsc.all_reduce_population_count(m)
        return lax.fori_loop(0, N_ATOMS // 16, per_vec, ptr_vec)

    cnt_vec = lax.fori_loop(0, APW, per_row, jnp.zeros((16,), jnp.int32))

    base = wid * WCAP
    pltpu.sync_copy(jbuf_v, jidx_hbm.at[pl.ds(base, WCAP)])
    pltpu.sync_copy(d2buf_v, d2e_hbm.at[pl.ds(base, WCAP)])
    pltpu.sync_copy(dstbuf_v, dst_hbm.at[pl.ds(base, WCAP)])
    cnt_v[...] = cnt_vec
    pltpu.sync_copy(cnt_v.at[pl.ds(0, 8)], cnt_hbm.at[pl.ds(wid * 8, 8)])


@jax.jit
def _build_nbr(d2):
    mesh = plsc.VectorSubcoreMesh(core_axis_name="c", subcore_axis_name="s")
    f = pl.kernel(
        _nbr_body,
        out_type=[
            jax.ShapeDtypeStruct((NEDIR,), jnp.int32),
            jax.ShapeDtypeStruct((NEDIR,), jnp.float32),
            jax.ShapeDtypeStruct((NEDIR,), jnp.int32),
            jax.ShapeDtypeStruct((NW * 8,), jnp.int32),
        ],
        mesh=mesh,
        compiler_params=pltpu.CompilerParams(needs_layout_passes=False),
        scratch_types=[
            pltpu.VMEM((N_ATOMS,), jnp.float32),
            pltpu.VMEM((WCAP,), jnp.int32),
            pltpu.VMEM((WCAP,), jnp.float32),
            pltpu.VMEM((WCAP,), jnp.int32),
            pltpu.VMEM((16,), jnp.int32),
        ],
    )
    return f(d2)


def _edge_mlp_body(ea_ref, w1_ref, b1_ref, w2_ref, b2_ref, h_ref):
    t = jnp.dot(ea_ref[...], w1_ref[...], preferred_element_type=jnp.float32)
    t = jax.nn.softplus(t + b1_ref[...]) - SHIFT
    h = jnp.dot(t.astype(jnp.bfloat16), w2_ref[...],
                preferred_element_type=jnp.float32)
    h = h + b2_ref[...]
    lo = jax.lax.bitcast_convert_type(
        h[:, :NFILT // 2].astype(jnp.bfloat16), jnp.uint16).astype(jnp.uint32)
    hi = jax.lax.bitcast_convert_type(
        h[:, NFILT // 2:].astype(jnp.bfloat16), jnp.uint16).astype(jnp.uint32)
    h_ref[...] = jax.lax.bitcast_convert_type(lo | (hi << 16), jnp.int32)


def _edge_mlp(ea_pad, w1p, b1, w2, b2):
    n_blk = NEDIR // EDGE_BLK
    return pl.pallas_call(
        _edge_mlp_body,
        grid=(n_blk,),
        in_specs=[
            pl.BlockSpec((EDGE_BLK, EA_PAD), lambda i: (i, 0)),
            pl.BlockSpec((EA_PAD, NFILT), lambda i: (0, 0)),
            pl.BlockSpec((1, NFILT), lambda i: (0, 0)),
            pl.BlockSpec((NFILT, NFILT), lambda i: (0, 0)),
            pl.BlockSpec((1, NFILT), lambda i: (0, 0)),
        ],
        cost_estimate=pl.CostEstimate(
            flops=2 * EDGE_BLK * NFILT * (EA_PAD + NFILT) * (NEDIR // EDGE_BLK),
            bytes_accessed=NEDIR * (2 * EA_PAD + 2 * NFILT),
            transcendentals=NEDIR * NFILT),
        out_specs=pl.BlockSpec((EDGE_BLK, NFILT // 2), lambda i: (i, 0)),
        out_shape=jax.ShapeDtypeStruct((NEDIR, NFILT // 2), jnp.int32),
    )(ea_pad, w1p, b1, w2, b2)


def _gms_body(xh_hbm, h_hbm, jidx_hbm, off_hbm, cnt_hbm, out_hbm,
              jidx0_v, off0_v, h0_v, gx0_v, jidx1_v, off1_v, h1_v, gx1_v,
              acc_v, cnt_v, xh_sh,
              sgx0, sh0, sgx1, sh1, sidx0, sidx1):
    cid = lax.axis_index("c")
    sid = lax.axis_index("s")
    wid = sid * NC + cid
    lanes = lax.iota(jnp.int32, 16)

    pltpu.sync_copy(xh_hbm.at[pl.ds(sid * (XH_ROWS // NS), XH_ROWS // NS)],
                    xh_sh.at[pl.ds(sid * (XH_ROWS // NS), XH_ROWS // NS)])

    def zacc(k, _):
        acc_v[pl.ds(k * 16, 16)] = jnp.zeros((16,), jnp.float32)
        return 0
    lax.fori_loop(0, 2 * ACC_ROWS * NFILT // 16, zacc, 0)
    plsc.subcore_barrier()

    pltpu.sync_copy(cnt_hbm, cnt_v)
    c0 = cnt_v[pl.ds(0, 16)]
    c1 = cnt_v[pl.ds(16, 16)]
    cnt = (jnp.sum(jnp.where(lanes == wid, c0, 0))
           + jnp.sum(jnp.where(lanes + 16 == wid, c1, 0)))
    npairs = jnp.maximum((cnt + 2 * CH - 1) // (2 * CH), 1)

    slots = ((jidx0_v, off0_v, h0_v, gx0_v, sgx0, sh0, sidx0),
             (jidx1_v, off1_v, h1_v, gx1_v, sgx1, sh1, sidx1))

    def issue_idx(ci, s):
        jv, ov, hv, gv, sgx, sh, sidx = slots[s]
        base = wid * WCAP + ci * CH
        pltpu.async_copy(jidx_hbm.at[pl.ds(base, CH)], jv, sidx)
        pltpu.async_copy(off_hbm.at[pl.ds(base * 16, CH * 16)], ov, sidx)

    def issue_data(ci, s):
        jv, ov, hv, gv, sgx, sh, sidx = slots[s]
        base = wid * WCAP + ci * CH
        pltpu.make_async_copy(jidx_hbm.at[pl.ds(base, CH)], jv, sidx).wait()
        pltpu.make_async_copy(off_hbm.at[pl.ds(base * 16, CH * 16)], ov,
                              sidx).wait()
        pltpu.async_copy(xh_sh.at[jv], gv, sgx)
        pltpu.async_copy(h_hbm.at[pl.ds(base, CH)], hv, sh)

    def wait_compute(ci, s, nidx):
        jv, ov, hv, gv, sgx, sh, sidx = slots[s]
        base = wid * WCAP + ci * CH
        pltpu.make_async_copy(xh_sh.at[jv], gv, sgx).wait()
        pltpu.make_async_copy(h_hbm.at[pl.ds(base, CH)], hv, sh).wait()

        @plsc.parallel_loop(0, CH, 1, unroll=2)
        def per_edge(e):
            offrow = ov[pl.ds(e * 16, 16)]
            for g in range(NFILT // 32):
                hw = hv[e, pl.ds(g * 16, 16)]
                gw = gv[e, pl.ds(g * 16, 16)]
                ha = plsc.bitcast(hw << 16, jnp.float32)
                hc = plsc.bitcast(hw & jnp.int32(-65536), jnp.float32)
                ga = plsc.bitcast(gw << 16, jnp.float32)
                gc = plsc.bitcast(gw & jnp.int32(-65536), jnp.float32)
                idx = offrow + (g * 16) + lanes
                plsc.addupdate_scatter(acc_v, [idx], ha * ga)
                plsc.addupdate_scatter(acc_v, [idx + NFILT // 2], hc * gc)

        @pl.when(nidx < 2 * npairs)
        def _():
            issue_idx(nidx, s)

    issue_idx(0, 0)
    issue_idx(1, 1)
    issue_data(0, 0)

    def pair(i, _):
        issue_data(2 * i + 1, 1)
        wait_compute(2 * i, 0, 2 * i + 2)

        @pl.when(i < npairs - 1)
        def _():
            issue_data(2 * i + 2, 0)
        wait_compute(2 * i + 1, 1, 2 * i + 3)
        return 0
    lax.fori_loop(0, npairs, pair, 0)

    def macc(k, _):
        sl = pl.ds(k * 16, 16)
        sl2 = pl.ds(ACC_ROWS * NFILT + k * 16, 16)
        acc_v[sl] = acc_v[sl] + acc_v[sl2]
        return 0
    lax.fori_loop(0, APW * NFILT // 16, macc, 0)
    pltpu.sync_copy(acc_v.at[pl.ds(0, APW * NFILT)],
                    out_hbm.at[pl.ds(wid * APW * NFILT, APW * NFILT)])


@jax.jit
def _gather_mul_acc(xh_pad, h, jidx, off_rep, cnts):
    mesh = plsc.VectorSubcoreMesh(core_axis_name="c", subcore_axis_name="s")
    f = pl.kernel(
        _gms_body,
        out_type=jax.ShapeDtypeStruct((N_ATOMS * NFILT,), jnp.float32),
        mesh=mesh,
        compiler_params=pltpu.CompilerParams(needs_layout_passes=False),
        scratch_types=[
            pltpu.VMEM((CH,), jnp.int32),
            pltpu.VMEM((CH * 16,), jnp.int32),
            pltpu.VMEM((CH, NFILT // 2), jnp.int32),
            pltpu.VMEM((CH, NFILT // 2), jnp.int32),
            pltpu.VMEM((CH,), jnp.int32),
            pltpu.VMEM((CH * 16,), jnp.int32),
            pltpu.VMEM((CH, NFILT // 2), jnp.int32),
            pltpu.VMEM((CH, NFILT // 2), jnp.int32),
            pltpu.VMEM((2 * ACC_ROWS * NFILT,), jnp.float32),
            pltpu.VMEM((NW,), jnp.int32),
            pltpu.VMEM_SHARED((XH_ROWS, NFILT // 2), jnp.int32),
            pltpu.SemaphoreType.DMA,
            pltpu.SemaphoreType.DMA,
            pltpu.SemaphoreType.DMA,
            pltpu.SemaphoreType.DMA,
            pltpu.SemaphoreType.DMA,
            pltpu.SemaphoreType.DMA,
        ],
    )
    return f(xh_pad, h, jidx, off_rep, cnts).reshape(N_ATOMS, NFILT)


def kernel(pos, z, cell, params):
    p = pos.astype(jnp.float32)
    c = cell.astype(jnp.float32)
    inv = jnp.linalg.inv(c)

    disp = p[:, None, :] - p[None, :, :]
    frac = disp @ inv
    frac = frac - jnp.round(frac)
    disp = frac @ c
    d2 = (disp ** 2).sum(-1)

    jidx, d2e, dstl, cnt8 = _build_nbr(d2)
    cnts = cnt8.reshape(NW, 8)[:, 0]
    off_rep = jnp.broadcast_to(dstl[:, None], (NEDIR, 16)).reshape(-1)

    dist = jnp.sqrt(d2e)
    offsets = jnp.linspace(0.0, R_CUT, N_GAUSS)
    width = offsets[1] - offsets[0]
    coeff = -0.5 / (width ** 2)
    ea = jnp.exp(coeff * (dist[:, None] - offsets[None, :]) ** 2)
    ea_pad = jnp.zeros((NEDIR, EA_PAD), jnp.float32).at[:, :N_GAUSS].set(ea)

    x = jnp.take(params['emb'], z, axis=0)
    hs = []
    for blk in params['blocks']:
        w1p = jnp.zeros((EA_PAD, NFILT), jnp.float32).at[:N_GAUSS].set(
            blk['mlp1']['w'])
        hs.append(_edge_mlp(ea_pad, w1p, blk['mlp1']['b'][None, :],
                            blk['mlp2']['w'].astype(jnp.bfloat16),
                            blk['mlp2']['b'][None, :]))
    for h_bits, blk in zip(hs, params['blocks']):
        xh = x @ blk['lin1']['w']
        xlo = jax.lax.bitcast_convert_type(
            xh[:, :NFILT // 2].astype(jnp.bfloat16),
            jnp.uint16).astype(jnp.uint32)
        xhi = jax.lax.bitcast_convert_type(
            xh[:, NFILT // 2:].astype(jnp.bfloat16),
            jnp.uint16).astype(jnp.uint32)
        xw = jax.lax.bitcast_convert_type(xlo | (xhi << 16), jnp.int32)
        xh_bits = jnp.zeros((XH_ROWS, NFILT // 2),
                            jnp.int32).at[:N_ATOMS].set(xw)
        aggr = _gather_mul_acc(xh_bits, h_bits, jidx, off_rep, cnts)
        y = aggr @ blk['lin2']['w'] + blk['lin2']['b']
        y = _ssp(y)
        y = y @ blk['lin']['w'] + blk['lin']['b']
        x = x + y
    o = _ssp(x @ params['out1']['w'] + params['out1']['b'])
    o = o @ params['out2']['w'] + params['out2']['b']
    return jnp.sum(o)


# speedup vs baseline: 1.0278x; 1.0278x over previous
import functools

import jax
import jax.numpy as jnp
import numpy as np
from jax import lax
from jax.experimental import pallas as pl
from jax.experimental.pallas import tpu as pltpu
from jax.experimental.pallas import tpu_sc as plsc

N_ATOMS = 2048
BOX = 24.0
R_CUT = 5.0
N_GAUSS = 50
HID = 256
NFILT = 256
SHIFT = float(np.log(2.0))

EA_PAD = 64
EDGE_BLK = 2048

NC = 2
NS = 16
NW = NC * NS
APW = N_ATOMS // NW
WCAP = 6144
NEDIR = NW * WCAP
CH = 128
ACC_ROWS = APW + 8
TRASH_XH = N_ATOMS
XH_ROWS = N_ATOMS + 128

R2CUT = R_CUT * R_CUT


def _ssp(x):
    return jax.nn.softplus(x) - SHIFT


def _nbr_body(d2_hbm, jidx_hbm, d2e_hbm, dst_hbm, cnt_hbm,
              d2row_v, jbuf_v, d2buf_v, dstbuf_v, cnt_v):
    cid = lax.axis_index("c")
    sid = lax.axis_index("s")
    wid = sid * NC + cid
    lanes = lax.iota(jnp.int32, 16)

    def pre(k, _):
        sl = pl.ds(k * 16, 16)
        jbuf_v[sl] = jnp.full((16,), TRASH_XH, jnp.int32)
        d2buf_v[sl] = jnp.zeros((16,), jnp.float32)
        dstbuf_v[sl] = jnp.full((16,), APW * NFILT, jnp.int32)
        return 0
    lax.fori_loop(0, WCAP // 16, pre, 0)

    def per_row(a, ptr_vec):
        i = wid * APW + a
        pltpu.sync_copy(d2_hbm.at[i], d2row_v)

        def per_vec(v, ptr_vec):
            d2v = d2row_v[pl.ds(v * 16, 16)]
            jv = lanes + v * 16
            m = (d2v < R2CUT) & (jv != i)
            pos = ptr_vec + plsc.cumsum(m.astype(jnp.int32)) - 1
            plsc.store_scatter(jbuf_v, [pos], jv, mask=m)
            plsc.store_scatter(d2buf_v, [pos], d2v, mask=m)
            plsc.store_scatter(dstbuf_v, [pos],
                               jnp.full((16,), a * NFILT, jnp.int32),
                               mask=m)
            return ptr_vec + plsc.all_reduce_population_count(m)
        return lax.fori_loop(0, N_ATOMS // 16, per_vec, ptr_vec)

    cnt_vec = lax.fori_loop(0, APW, per_row, jnp.zeros((16,), jnp.int32))

    base = wid * WCAP
    pltpu.sync_copy(jbuf_v, jidx_hbm.at[pl.ds(base, WCAP)])
    pltpu.sync_copy(d2buf_v, d2e_hbm.at[pl.ds(base, WCAP)])
    pltpu.sync_copy(dstbuf_v, dst_hbm.at[pl.ds(base, WCAP)])
    cnt_v[...] = cnt_vec
    pltpu.sync_copy(cnt_v.at[pl.ds(0, 8)], cnt_hbm.at[pl.ds(wid * 8, 8)])


@jax.jit
def _build_nbr(d2):
    mesh = plsc.VectorSubcoreMesh(core_axis_name="c", subcore_axis_name="s")
    f = pl.kernel(
        _nbr_body,
        out_type=[
            jax.ShapeDtypeStruct((NEDIR,), jnp.int32),
            jax.ShapeDtypeStruct((NEDIR,), jnp.float32),
            jax.ShapeDtypeStruct((NEDIR,), jnp.int32),
            jax.ShapeDtypeStruct((NW * 8,), jnp.int32),
        ],
        mesh=mesh,
        compiler_params=pltpu.CompilerParams(needs_layout_passes=False),
        scratch_types=[
            pltpu.VMEM((N_ATOMS,), jnp.float32),
            pltpu.VMEM((WCAP,), jnp.int32),
            pltpu.VMEM((WCAP,), jnp.float32),
            pltpu.VMEM((WCAP,), jnp.int32),
            pltpu.VMEM((16,), jnp.int32),
        ],
    )
    return f(d2)


def _edge_mlp_body(ea_ref, w1_ref, b1_ref, w2_ref, b2_ref, h_ref):
    t = jnp.dot(ea_ref[...].astype(jnp.float32), w1_ref[...],
                preferred_element_type=jnp.float32)
    t = jax.nn.softplus(t + b1_ref[...]) - SHIFT
    h = jnp.dot(t.astype(jnp.bfloat16), w2_ref[...],
                preferred_element_type=jnp.float32)
    h = h + b2_ref[...]
    lo = jax.lax.bitcast_convert_type(
        h[:, :NFILT // 2].astype(jnp.bfloat16), jnp.uint16).astype(jnp.uint32)
    hi = jax.lax.bitcast_convert_type(
        h[:, NFILT // 2:].astype(jnp.bfloat16), jnp.uint16).astype(jnp.uint32)
    h_ref[...] = jax.lax.bitcast_convert_type(lo | (hi << 16), jnp.int32)


def _edge_mlp(ea_pad, w1p, b1, w2, b2):
    n_blk = NEDIR // EDGE_BLK
    return pl.pallas_call(
        _edge_mlp_body,
        grid=(n_blk,),
        in_specs=[
            pl.BlockSpec((EDGE_BLK, EA_PAD), lambda i: (i, 0)),
            pl.BlockSpec((EA_PAD, NFILT), lambda i: (0, 0)),
            pl.BlockSpec((1, NFILT), lambda i: (0, 0)),
            pl.BlockSpec((NFILT, NFILT), lambda i: (0, 0)),
            pl.BlockSpec((1, NFILT), lambda i: (0, 0)),
        ],
        cost_estimate=pl.CostEstimate(
            flops=2 * EDGE_BLK * NFILT * (EA_PAD + NFILT) * (NEDIR // EDGE_BLK),
            bytes_accessed=NEDIR * (2 * EA_PAD + 2 * NFILT),
            transcendentals=NEDIR * NFILT),
        out_specs=pl.BlockSpec((EDGE_BLK, NFILT // 2), lambda i: (i, 0)),
        out_shape=jax.ShapeDtypeStruct((NEDIR, NFILT // 2), jnp.int32),
    )(ea_pad, w1p, b1, w2, b2)


def _gms_body(xh_hbm, h_hbm, jidx_hbm, off_hbm, cnt_hbm, out_hbm,
              jidx0_v, off0_v, h0_v, gx0_v, jidx1_v, off1_v, h1_v, gx1_v,
              acc_v, cnt_v, xh_sh,
              sgx0, sh0, sgx1, sh1, sidx0, sidx1):
    cid = lax.axis_index("c")
    sid = lax.axis_index("s")
    wid = sid * NC + cid
    lanes = lax.iota(jnp.int32, 16)

    pltpu.sync_copy(xh_hbm.at[pl.ds(sid * (XH_ROWS // NS), XH_ROWS // NS)],
                    xh_sh.at[pl.ds(sid * (XH_ROWS // NS), XH_ROWS // NS)])

    def zacc(k, _):
        acc_v[pl.ds(k * 16, 16)] = jnp.zeros((16,), jnp.float32)
        return 0
    lax.fori_loop(0, 2 * ACC_ROWS * NFILT // 16, zacc, 0)
    plsc.subcore_barrier()

    pltpu.sync_copy(cnt_hbm, cnt_v)
    c0 = cnt_v[pl.ds(0, 16)]
    c1 = cnt_v[pl.ds(16, 16)]
    cnt = (jnp.sum(jnp.where(lanes == wid, c0, 0))
           + jnp.sum(jnp.where(lanes + 16 == wid, c1, 0)))
    npairs = jnp.maximum((cnt + 2 * CH - 1) // (2 * CH), 1)

    slots = ((jidx0_v, off0_v, h0_v, gx0_v, sgx0, sh0, sidx0),
             (jidx1_v, off1_v, h1_v, gx1_v, sgx1, sh1, sidx1))

    def issue_idx(ci, s):
        jv, ov, hv, gv, sgx, sh, sidx = slots[s]
        base = wid * WCAP + ci * CH
        pltpu.async_copy(jidx_hbm.at[pl.ds(base, CH)], jv, sidx)
        pltpu.async_copy(off_hbm.at[pl.ds(base * 16, CH * 16)], ov, sidx)

    def issue_data(ci, s):
        jv, ov, hv, gv, sgx, sh, sidx = slots[s]
        base = wid * WCAP + ci * CH
        pltpu.make_async_copy(jidx_hbm.at[pl.ds(base, CH)], jv, sidx).wait()
        pltpu.make_async_copy(off_hbm.at[pl.ds(base * 16, CH * 16)], ov,
                              sidx).wait()
        pltpu.async_copy(xh_sh.at[jv], gv, sgx)
        pltpu.async_copy(h_hbm.at[pl.ds(base, CH)], hv, sh)

    def wait_compute(ci, s, nidx):
        jv, ov, hv, gv, sgx, sh, sidx = slots[s]
        base = wid * WCAP + ci * CH
        pltpu.make_async_copy(xh_sh.at[jv], gv, sgx).wait()
        pltpu.make_async_copy(h_hbm.at[pl.ds(base, CH)], hv, sh).wait()

        @plsc.parallel_loop(0, CH, 1, unroll=2)
        def per_edge(e):
            offrow = ov[pl.ds(e * 16, 16)]
            for g in range(NFILT // 32):
                hw = hv[e, pl.ds(g * 16, 16)]
                gw = gv[e, pl.ds(g * 16, 16)]
                ha = plsc.bitcast(hw << 16, jnp.float32)
                hc = plsc.bitcast(hw & jnp.int32(-65536), jnp.float32)
                ga = plsc.bitcast(gw << 16, jnp.float32)
                gc = plsc.bitcast(gw & jnp.int32(-65536), jnp.float32)
                idx = offrow + (g * 16) + lanes
                plsc.addupdate_scatter(acc_v, [idx], ha * ga)
                plsc.addupdate_scatter(acc_v, [idx + NFILT // 2], hc * gc)

        @pl.when(nidx < 2 * npairs)
        def _():
            issue_idx(nidx, s)

    issue_idx(0, 0)
    issue_idx(1, 1)
    issue_data(0, 0)

    def pair(i, _):
        issue_data(2 * i + 1, 1)
        wait_compute(2 * i, 0, 2 * i + 2)

        @pl.when(i < npairs - 1)
        def _():
            issue_data(2 * i + 2, 0)
        wait_compute(2 * i + 1, 1, 2 * i + 3)
        return 0
    lax.fori_loop(0, npairs, pair, 0)

    def macc(k, _):
        sl = pl.ds(k * 16, 16)
        sl2 = pl.ds(ACC_ROWS * NFILT + k * 16, 16)
        acc_v[sl] = acc_v[sl] + acc_v[sl2]
        return 0
    lax.fori_loop(0, APW * NFILT // 16, macc, 0)
    pltpu.sync_copy(acc_v.at[pl.ds(0, APW * NFILT)],
                    out_hbm.at[pl.ds(wid * APW * NFILT, APW * NFILT)])


@jax.jit
def _gather_mul_acc(xh_pad, h, jidx, off_rep, cnts):
    mesh = plsc.VectorSubcoreMesh(core_axis_name="c", subcore_axis_name="s")
    f = pl.kernel(
        _gms_body,
        out_type=jax.ShapeDtypeStruct((N_ATOMS * NFILT,), jnp.float32),
        mesh=mesh,
        compiler_params=pltpu.CompilerParams(needs_layout_passes=False),
        scratch_types=[
            pltpu.VMEM((CH,), jnp.int32),
            pltpu.VMEM((CH * 16,), jnp.int32),
            pltpu.VMEM((CH, NFILT // 2), jnp.int32),
            pltpu.VMEM((CH, NFILT // 2), jnp.int32),
            pltpu.VMEM((CH,), jnp.int32),
            pltpu.VMEM((CH * 16,), jnp.int32),
            pltpu.VMEM((CH, NFILT // 2), jnp.int32),
            pltpu.VMEM((CH, NFILT // 2), jnp.int32),
            pltpu.VMEM((2 * ACC_ROWS * NFILT,), jnp.float32),
            pltpu.VMEM((NW,), jnp.int32),
            pltpu.VMEM_SHARED((XH_ROWS, NFILT // 2), jnp.int32),
            pltpu.SemaphoreType.DMA,
            pltpu.SemaphoreType.DMA,
            pltpu.SemaphoreType.DMA,
            pltpu.SemaphoreType.DMA,
            pltpu.SemaphoreType.DMA,
            pltpu.SemaphoreType.DMA,
        ],
    )
    return f(xh_pad, h, jidx, off_rep, cnts).reshape(N_ATOMS, NFILT)


def kernel(pos, z, cell, params):
    p = pos.astype(jnp.float32)
    c = cell.astype(jnp.float32)
    inv = jnp.linalg.inv(c)

    disp = p[:, None, :] - p[None, :, :]
    frac = disp @ inv
    frac = frac - jnp.round(frac)
    disp = frac @ c
    d2 = (disp ** 2).sum(-1)

    jidx, d2e, dstl, cnt8 = _build_nbr(d2)
    cnts = cnt8.reshape(NW, 8)[:, 0]
    off_rep = jnp.broadcast_to(dstl[:, None], (NEDIR, 16)).reshape(-1)

    dist = jnp.sqrt(d2e)
    offsets = jnp.linspace(0.0, R_CUT, N_GAUSS)
    width = offsets[1] - offsets[0]
    coeff = -0.5 / (width ** 2)
    ea = jnp.exp(coeff * (dist[:, None] - offsets[None, :]) ** 2)
    ea_pad = jnp.zeros((NEDIR, EA_PAD),
                       jnp.bfloat16).at[:, :N_GAUSS].set(
                           ea.astype(jnp.bfloat16))

    x = jnp.take(params['emb'], z, axis=0)
    hs = []
    for blk in params['blocks']:
        w1p = jnp.zeros((EA_PAD, NFILT), jnp.float32).at[:N_GAUSS].set(
            blk['mlp1']['w'])
        hs.append(_edge_mlp(ea_pad, w1p, blk['mlp1']['b'][None, :],
                            blk['mlp2']['w'].astype(jnp.bfloat16),
                            blk['mlp2']['b'][None, :]))
    for h_bits, blk in zip(hs, params['blocks']):
        xh = x @ blk['lin1']['w']
        xlo = jax.lax.bitcast_convert_type(
            xh[:, :NFILT // 2].astype(jnp.bfloat16),
            jnp.uint16).astype(jnp.uint32)
        xhi = jax.lax.bitcast_convert_type(
            xh[:, NFILT // 2:].astype(jnp.bfloat16),
            jnp.uint16).astype(jnp.uint32)
        xw = jax.lax.bitcast_convert_type(xlo | (xhi << 16), jnp.int32)
        xh_bits = jnp.zeros((XH_ROWS, NFILT // 2),
                            jnp.int32).at[:N_ATOMS].set(xw)
        aggr = _gather_mul_acc(xh_bits, h_bits, jidx, off_rep, cnts)
        y = aggr @ blk['lin2']['w'] + blk['lin2']['b']
        y = _ssp(y)
        y = y @ blk['lin']['w'] + blk['lin']['b']
        x = x + y
    o = _ssp(x @ params['out1']['w'] + params['out1']['b'])
    o = o @ params['out2']['w'] + params['out2']['b']
    return jnp.sum(o)


# speedup vs baseline: 1.0406x; 1.0125x over previous
import functools

import jax
import jax.numpy as jnp
import numpy as np
from jax import lax
from jax.experimental import pallas as pl
from jax.experimental.pallas import tpu as pltpu
from jax.experimental.pallas import tpu_sc as plsc

N_ATOMS = 2048
BOX = 24.0
R_CUT = 5.0
N_GAUSS = 50
HID = 256
NFILT = 256
SHIFT = float(np.log(2.0))

EA_PAD = 64
EDGE_BLK = 2048

NC = 2
NS = 16
NW = NC * NS
APW = N_ATOMS // NW
WCAP = 6144
NEDIR = NW * WCAP
CH = 128
ACC_ROWS = APW + 8
TRASH_XH = N_ATOMS
XH_ROWS = N_ATOMS + 128

R2CUT = R_CUT * R_CUT


def _ssp(x):
    return jax.nn.softplus(x) - SHIFT


def _nbr_body(d2_hbm, jidx_hbm, d2e_hbm, dst_hbm, cnt_hbm,
              d2row_v, jbuf_v, d2buf_v, dstbuf_v, cnt_v):
    cid = lax.axis_index("c")
    sid = lax.axis_index("s")
    wid = sid * NC + cid
    lanes = lax.iota(jnp.int32, 16)

    def pre(k, _):
        sl = pl.ds(k * 16, 16)
        jbuf_v[sl] = jnp.full((16,), TRASH_XH, jnp.int32)
        d2buf_v[sl] = jnp.zeros((16,), jnp.float32)
        dstbuf_v[sl] = jnp.full((16,), APW * NFILT, jnp.int32)
        return 0
    lax.fori_loop(0, WCAP // 16, pre, 0)

    def per_row(a, ptrs):
        i = wid * APW + a
        pltpu.sync_copy(d2_hbm.at[i], d2row_v)

        def per_vec(v, ptrs):
            ptr_a, ptr_b = ptrs
            dva = d2row_v[pl.ds(v * 16, 16)]
            dvb = d2row_v[pl.ds((v + 64) * 16, 16)]
            ja = lanes + v * 16
            jb = lanes + (v + 64) * 16
            ma = (dva < R2CUT) & (ja != i)
            mb = (dvb < R2CUT) & (jb != i)
            pos_a = ptr_a + plsc.cumsum(ma.astype(jnp.int32)) - 1
            pos_b = ptr_b + plsc.cumsum(mb.astype(jnp.int32)) - 1
            dst = jnp.full((16,), a * NFILT, jnp.int32)
            plsc.store_scatter(jbuf_v, [pos_a], ja, mask=ma)
            plsc.store_scatter(jbuf_v, [pos_b], jb, mask=mb)
            plsc.store_scatter(d2buf_v, [pos_a], dva, mask=ma)
            plsc.store_scatter(d2buf_v, [pos_b], dvb, mask=mb)
            plsc.store_scatter(dstbuf_v, [pos_a], dst, mask=ma)
            plsc.store_scatter(dstbuf_v, [pos_b], dst, mask=mb)
            return (ptr_a + plsc.all_reduce_population_count(ma),
                    ptr_b + plsc.all_reduce_population_count(mb))
        return lax.fori_loop(0, N_ATOMS // 32, per_vec, ptrs)

    half = jnp.full((16,), WCAP // 2, jnp.int32)
    cnt_a, cnt_b = lax.fori_loop(
        0, APW, per_row, (jnp.zeros((16,), jnp.int32), half))

    base = wid * WCAP
    pltpu.sync_copy(jbuf_v, jidx_hbm.at[pl.ds(base, WCAP)])
    pltpu.sync_copy(d2buf_v, d2e_hbm.at[pl.ds(base, WCAP)])
    pltpu.sync_copy(dstbuf_v, dst_hbm.at[pl.ds(base, WCAP)])
    cnt_v[...] = jnp.where(lanes < 8, cnt_a, cnt_b - (WCAP // 2))
    pltpu.sync_copy(cnt_v, cnt_hbm.at[pl.ds(wid * 16, 16)])


@jax.jit
def _build_nbr(d2):
    mesh = plsc.VectorSubcoreMesh(core_axis_name="c", subcore_axis_name="s")
    f = pl.kernel(
        _nbr_body,
        out_type=[
            jax.ShapeDtypeStruct((NEDIR,), jnp.int32),
            jax.ShapeDtypeStruct((NEDIR,), jnp.float32),
            jax.ShapeDtypeStruct((NEDIR,), jnp.int32),
            jax.ShapeDtypeStruct((NW * 16,), jnp.int32),
        ],
        mesh=mesh,
        compiler_params=pltpu.CompilerParams(needs_layout_passes=False),
        scratch_types=[
            pltpu.VMEM((N_ATOMS,), jnp.float32),
            pltpu.VMEM((WCAP,), jnp.int32),
            pltpu.VMEM((WCAP,), jnp.float32),
            pltpu.VMEM((WCAP,), jnp.int32),
            pltpu.VMEM((16,), jnp.int32),
        ],
    )
    return f(d2)


def _edge_mlp_body(ea_ref, w1_ref, b1_ref, w2_ref, b2_ref, h_ref):
    t = jnp.dot(ea_ref[...].astype(jnp.float32), w1_ref[...],
                preferred_element_type=jnp.float32)
    t = jax.nn.softplus(t + b1_ref[...]) - SHIFT
    h = jnp.dot(t.astype(jnp.bfloat16), w2_ref[...],
                preferred_element_type=jnp.float32)
    h = h + b2_ref[...]
    lo = jax.lax.bitcast_convert_type(
        h[:, :NFILT // 2].astype(jnp.bfloat16), jnp.uint16).astype(jnp.uint32)
    hi = jax.lax.bitcast_convert_type(
        h[:, NFILT // 2:].astype(jnp.bfloat16), jnp.uint16).astype(jnp.uint32)
    h_ref[...] = jax.lax.bitcast_convert_type(lo | (hi << 16), jnp.int32)


def _edge_mlp(ea_pad, w1p, b1, w2, b2):
    n_blk = NEDIR // EDGE_BLK
    return pl.pallas_call(
        _edge_mlp_body,
        grid=(n_blk,),
        in_specs=[
            pl.BlockSpec((EDGE_BLK, EA_PAD), lambda i: (i, 0)),
            pl.BlockSpec((EA_PAD, NFILT), lambda i: (0, 0)),
            pl.BlockSpec((1, NFILT), lambda i: (0, 0)),
            pl.BlockSpec((NFILT, NFILT), lambda i: (0, 0)),
            pl.BlockSpec((1, NFILT), lambda i: (0, 0)),
        ],
        cost_estimate=pl.CostEstimate(
            flops=2 * EDGE_BLK * NFILT * (EA_PAD + NFILT) * (NEDIR // EDGE_BLK),
            bytes_accessed=NEDIR * (2 * EA_PAD + 2 * NFILT),
            transcendentals=NEDIR * NFILT),
        out_specs=pl.BlockSpec((EDGE_BLK, NFILT // 2), lambda i: (i, 0)),
        out_shape=jax.ShapeDtypeStruct((NEDIR, NFILT // 2), jnp.int32),
    )(ea_pad, w1p, b1, w2, b2)


def _gms_body(xh_hbm, h_hbm, jidx_hbm, off_hbm, cnt_hbm, out_hbm,
              jidx0_v, off0_v, h0_v, gx0_v, jidx1_v, off1_v, h1_v, gx1_v,
              acc_v, cnt_v, xh_sh,
              sgx0, sh0, sgx1, sh1, sidx0, sidx1):
    cid = lax.axis_index("c")
    sid = lax.axis_index("s")
    wid = sid * NC + cid
    lanes = lax.iota(jnp.int32, 16)

    pltpu.sync_copy(xh_hbm.at[pl.ds(sid * (XH_ROWS // NS), XH_ROWS // NS)],
                    xh_sh.at[pl.ds(sid * (XH_ROWS // NS), XH_ROWS // NS)])

    def zacc(k, _):
        acc_v[pl.ds(k * 16, 16)] = jnp.zeros((16,), jnp.float32)
        return 0
    lax.fori_loop(0, 2 * ACC_ROWS * NFILT // 16, zacc, 0)
    plsc.subcore_barrier()

    pltpu.sync_copy(cnt_hbm, cnt_v)

    def cnt_at(pos):
        acc = jnp.int32(0)
        for k in range(2 * NW // 16):
            ck = cnt_v[pl.ds(k * 16, 16)]
            acc = acc + jnp.sum(jnp.where(lanes + k * 16 == pos, ck, 0))
        return acc

    slots = ((jidx0_v, off0_v, h0_v, gx0_v, sgx0, sh0, sidx0),
             (jidx1_v, off1_v, h1_v, gx1_v, sgx1, sh1, sidx1))

    def run_region(rbase, cnt):
        npairs = jnp.maximum((cnt + 2 * CH - 1) // (2 * CH), 1)

        def issue_idx(ci, s):
            jv, ov, hv, gv, sgx, sh, sidx = slots[s]
            base = rbase + ci * CH
            pltpu.async_copy(jidx_hbm.at[pl.ds(base, CH)], jv, sidx)
            pltpu.async_copy(off_hbm.at[pl.ds(base * 16, CH * 16)], ov, sidx)

        def issue_data(ci, s):
            jv, ov, hv, gv, sgx, sh, sidx = slots[s]
            base = rbase + ci * CH
            pltpu.make_async_copy(jidx_hbm.at[pl.ds(base, CH)], jv,
                                  sidx).wait()
            pltpu.make_async_copy(off_hbm.at[pl.ds(base * 16, CH * 16)], ov,
                                  sidx).wait()
            pltpu.async_copy(xh_sh.at[jv], gv, sgx)
            pltpu.async_copy(h_hbm.at[pl.ds(base, CH)], hv, sh)

        def wait_compute(ci, s, nidx):
            jv, ov, hv, gv, sgx, sh, sidx = slots[s]
            base = rbase + ci * CH
            pltpu.make_async_copy(xh_sh.at[jv], gv, sgx).wait()
            pltpu.make_async_copy(h_hbm.at[pl.ds(base, CH)], hv, sh).wait()

            @plsc.parallel_loop(0, CH, 1, unroll=2)
            def per_edge(e):
                offrow = ov[pl.ds(e * 16, 16)]
                for g in range(NFILT // 32):
                    hw = hv[e, pl.ds(g * 16, 16)]
                    gw = gv[e, pl.ds(g * 16, 16)]
                    ha = plsc.bitcast(hw << 16, jnp.float32)
                    hc = plsc.bitcast(hw & jnp.int32(-65536), jnp.float32)
                    ga = plsc.bitcast(gw << 16, jnp.float32)
                    gc = plsc.bitcast(gw & jnp.int32(-65536), jnp.float32)
                    idx = offrow + (g * 16) + lanes
                    plsc.addupdate_scatter(acc_v, [idx], ha * ga)
                    plsc.addupdate_scatter(acc_v, [idx + NFILT // 2],
                                           hc * gc)

            @pl.when(nidx < 2 * npairs)
            def _():
                issue_idx(nidx, s)

        issue_idx(0, 0)
        issue_idx(1, 1)
        issue_data(0, 0)

        def pair(i, _):
            issue_data(2 * i + 1, 1)
            wait_compute(2 * i, 0, 2 * i + 2)

            @pl.when(i < npairs - 1)
            def _():
                issue_data(2 * i + 2, 0)
            wait_compute(2 * i + 1, 1, 2 * i + 3)
            return 0
        lax.fori_loop(0, npairs, pair, 0)

    run_region(wid * WCAP, cnt_at(2 * wid))
    run_region(wid * WCAP + WCAP // 2, cnt_at(2 * wid + 1))

    def macc(k, _):
        sl = pl.ds(k * 16, 16)
        sl2 = pl.ds(ACC_ROWS * NFILT + k * 16, 16)
        acc_v[sl] = acc_v[sl] + acc_v[sl2]
        return 0
    lax.fori_loop(0, APW * NFILT // 16, macc, 0)
    pltpu.sync_copy(acc_v.at[pl.ds(0, APW * NFILT)],
                    out_hbm.at[pl.ds(wid * APW * NFILT, APW * NFILT)])


@jax.jit
def _gather_mul_acc(xh_pad, h, jidx, off_rep, cnts):
    mesh = plsc.VectorSubcoreMesh(core_axis_name="c", subcore_axis_name="s")
    f = pl.kernel(
        _gms_body,
        out_type=jax.ShapeDtypeStruct((N_ATOMS * NFILT,), jnp.float32),
        mesh=mesh,
        compiler_params=pltpu.CompilerParams(needs_layout_passes=False),
        scratch_types=[
            pltpu.VMEM((CH,), jnp.int32),
            pltpu.VMEM((CH * 16,), jnp.int32),
            pltpu.VMEM((CH, NFILT // 2), jnp.int32),
            pltpu.VMEM((CH, NFILT // 2), jnp.int32),
            pltpu.VMEM((CH,), jnp.int32),
            pltpu.VMEM((CH * 16,), jnp.int32),
            pltpu.VMEM((CH, NFILT // 2), jnp.int32),
            pltpu.VMEM((CH, NFILT // 2), jnp.int32),
            pltpu.VMEM((2 * ACC_ROWS * NFILT,), jnp.float32),
            pltpu.VMEM((2 * NW,), jnp.int32),
            pltpu.VMEM_SHARED((XH_ROWS, NFILT // 2), jnp.int32),
            pltpu.SemaphoreType.DMA,
            pltpu.SemaphoreType.DMA,
            pltpu.SemaphoreType.DMA,
            pltpu.SemaphoreType.DMA,
            pltpu.SemaphoreType.DMA,
            pltpu.SemaphoreType.DMA,
        ],
    )
    return f(xh_pad, h, jidx, off_rep, cnts).reshape(N_ATOMS, NFILT)


def kernel(pos, z, cell, params):
    p = pos.astype(jnp.float32)
    c = cell.astype(jnp.float32)
    inv = jnp.linalg.inv(c)

    disp = p[:, None, :] - p[None, :, :]
    frac = disp @ inv
    frac = frac - jnp.round(frac)
    disp = frac @ c
    d2 = (disp ** 2).sum(-1)

    jidx, d2e, dstl, cnt8 = _build_nbr(d2)
    cnts = cnt8.reshape(NW, 2, 8)[:, :, 0].reshape(-1)
    off_rep = jnp.broadcast_to(dstl[:, None], (NEDIR, 16)).reshape(-1)

    dist = jnp.sqrt(d2e)
    offsets = jnp.linspace(0.0, R_CUT, N_GAUSS)
    width = offsets[1] - offsets[0]
    coeff = -0.5 / (width ** 2)
    ea = jnp.exp(coeff * (dist[:, None] - offsets[None, :]) ** 2)
    ea_pad = jnp.zeros((NEDIR, EA_PAD), jnp.float32).at[:, :N_GAUSS].set(ea)

    x = jnp.take(params['emb'], z, axis=0)
    hs = []
    for blk in params['blocks']:
        w1p = jnp.zeros((EA_PAD, NFILT), jnp.float32).at[:N_GAUSS].set(
            blk['mlp1']['w'])
        hs.append(_edge_mlp(ea_pad, w1p, blk['mlp1']['b'][None, :],
                            blk['mlp2']['w'].astype(jnp.bfloat16),
                            blk['mlp2']['b'][None, :]))
    for h_bits, blk in zip(hs, params['blocks']):
        xh = x @ blk['lin1']['w']
        xlo = jax.lax.bitcast_convert_type(
            xh[:, :NFILT // 2].astype(jnp.bfloat16),
            jnp.uint16).astype(jnp.uint32)
        xhi = jax.lax.bitcast_convert_type(
            xh[:, NFILT // 2:].astype(jnp.bfloat16),
            jnp.uint16).astype(jnp.uint32)
        xw = jax.lax.bitcast_convert_type(xlo | (xhi << 16), jnp.int32)
        xh_bits = jnp.zeros((XH_ROWS, NFILT // 2),
                            jnp.int32).at[:N_ATOMS].set(xw)
        aggr = _gather_mul_acc(xh_bits, h_bits, jidx, off_rep, cnts)
        y = aggr @ blk['lin2']['w'] + blk['lin2']['b']
        y = _ssp(y)
        y = y @ blk['lin']['w'] + blk['lin']['b']
        x = x + y
    o = _ssp(x @ params['out1']['w'] + params['out1']['b'])
    o = o @ params['out2']['w'] + params['out2']['b']
    return jnp.sum(o)
